# baseline (speedup 1.0000x reference)
import functools

import jax
import jax.numpy as jnp
from jax import lax
from jax.experimental import pallas as pl
from jax.experimental.pallas import tpu as pltpu
from jax.experimental.pallas import tpu_sc as plsc

DOC_LEN = 16384
EMB = 64
NUM_TILES = 32
ROWS_PER_TILE = DOC_LEN // NUM_TILES
CHUNK = 32
NCHUNK = ROWS_PER_TILE // CHUNK


def _sc_stage(x0, x1, emb1, emb2):
    mesh = plsc.VectorSubcoreMesh(core_axis_name="c", subcore_axis_name="s")

    @functools.partial(
        pl.kernel,
        out_type=(
            jax.ShapeDtypeStruct((NUM_TILES, EMB, 16), jnp.float32),
            jax.ShapeDtypeStruct((8, EMB), jnp.float32),
        ),
        mesh=mesh,
        compiler_params=pltpu.CompilerParams(needs_layout_passes=False),
        scratch_types=[
            pltpu.VMEM((ROWS_PER_TILE,), jnp.int32),
            pltpu.VMEM((ROWS_PER_TILE,), jnp.int32),
            pltpu.VMEM((2, CHUNK, 8, EMB), jnp.float32),
            pltpu.VMEM((EMB, 16), jnp.float32),
            pltpu.VMEM((16,), jnp.int32),
            pltpu.VMEM((8, EMB), jnp.float32),
            pltpu.SemaphoreType.DMA,
            pltpu.SemaphoreType.DMA,
        ],
    )
    def k(x0_hbm, x1_hbm, emb1_hbm, emb2_hbm, part_hbm, y2_hbm,
          idx_v, tidx_v, tiles_v, acc_v, cp_v, y2_v, sem, sem2):
        wid = lax.axis_index("s") * 2 + lax.axis_index("c")
        base = wid * ROWS_PER_TILE

        pltpu.sync_copy(x0_hbm.at[pl.ds(base, ROWS_PER_TILE)], idx_v)

        m8 = jnp.full((16,), -8, jnp.int32)
        for s in range(ROWS_PER_TILE // 16):
            tidx_v[pl.ds(s * 16, 16)] = (
                lax.bitwise_and(idx_v[pl.ds(s * 16, 16)], m8))

        zero = jnp.zeros((16,), jnp.float32)
        for c in range(EMB):
            acc_v[c, :] = zero

        @pl.when(wid == 0)
        def _():
            x1v = x1_hbm.at[pl.ds(0, 1)]
            pltpu.sync_copy(x1v, cp_v.at[pl.ds(0, 1)])
            t1 = cp_v[pl.ds(0, 16)][0] & (-8)
            pltpu.async_copy(
                emb2_hbm.at[pl.ds(pl.multiple_of(t1, 8), 8)],
                y2_v, sem2).wait()
            pltpu.sync_copy(y2_v, y2_hbm)

        lane = lax.iota(jnp.int32, 16)

        def fire(c, b):
            for g in range(CHUNK // 16):
                tv = tidx_v[pl.ds(c * CHUNK + g * 16, 16)]
                for j in range(16):
                    pltpu.async_copy(
                        emb1_hbm.at[pl.ds(pl.multiple_of(tv[j], 8), 8)],
                        tiles_v.at[b].at[g * 16 + j], sem)

        def drain(b):
            for j in range(CHUNK):
                pltpu.make_async_copy(
                    emb1_hbm.at[pl.ds(0, 8)], tiles_v.at[b].at[j],
                    sem).wait()

        def extract(c, b):
            for g in range(CHUNK // 16):
                iv = idx_v[pl.ds(c * CHUNK + g * 16, 16)]
                r = lax.bitwise_and(iv, jnp.full((16,), 7, jnp.int32))
                l = lane + g * 16
                for f in range(EMB):
                    v = plsc.load_gather(
                        tiles_v.at[b],
                        [l, r, jnp.full((16,), f, jnp.int32)])
                    plsc.addupdate(acc_v.at[f], v)

        fire(0, 0)
        fire(1, 1)

        def step(i, carry):
            for b in range(2):
                c = 2 * i + b
                drain(b)
                extract(c, b)

                @pl.when(c + 2 < NCHUNK)
                def _():
                    fire(c + 2, b)
            return carry

        lax.fori_loop(0, NCHUNK // 2, step, 0)

        pltpu.sync_copy(acc_v, part_hbm.at[wid])

    return k(x0, x1, emb1, emb2)


def _tc_stage(partials, y2tile, x1, x2, W1, b1, W2, b2):
    def body(x1_ref, p_ref, y2_ref, x2_ref, w1_ref, b1_ref, w2_ref, b2_ref,
             o_ref):
        s64 = jnp.sum(p_ref[...], axis=(0, 2)) * (1.0 / DOC_LEN)
        s = s64.reshape(1, EMB)
        y1 = lax.dot_general(s, w1_ref[...], (((1,), (1,)), ((), ())),
                             preferred_element_type=jnp.float32) + b1_ref[...]
        r = x1_ref[0] & 7
        y2 = y2_ref[pl.ds(r, 1), :]
        y3 = jax.nn.sigmoid(
            lax.dot_general(x2_ref[...], w2_ref[...], (((1,), (1,)), ((), ())),
                            preferred_element_type=jnp.float32) + b2_ref[...])
        t = y2 + y3
        o_ref[...] = jax.nn.sigmoid(jnp.sum(y1 * t, axis=1, keepdims=True))

    return pl.pallas_call(
        body,
        in_specs=[pl.BlockSpec(memory_space=pltpu.SMEM)]
        + [pl.BlockSpec()] * 7,
        out_shape=jax.ShapeDtypeStruct((1, 1), jnp.float32),
    )(x1, partials, y2tile, x2, W1, b1, W2, b2)


def kernel(x0, x1, x2, emb1, emb2, W1, b1, W2, b2):
    partials, y2tile = _sc_stage(x0, x1.astype(jnp.int32), emb1, emb2)
    out = _tc_stage(partials, y2tile, x1.astype(jnp.int32),
                    x2.reshape(1, EMB), W1, b1.reshape(1, EMB),
                    W2, b2.reshape(1, EMB))
    return out.reshape(())


# speedup vs baseline: 3.1838x; 3.1838x over previous
import functools

import jax
import jax.numpy as jnp
from jax import lax
from jax.experimental import pallas as pl
from jax.experimental.pallas import tpu as pltpu
from jax.experimental.pallas import tpu_sc as plsc

DOC_LEN = 16384
EMB = 64
NUM_WORDS = 1000000
PAD_WORDS = 1 << 20
NUM_CP = 100000
NUM_TILES = 32
ROWS_PER_TILE = DOC_LEN // NUM_TILES
PER_TILE_WORDS = PAD_WORDS // 16
BLK = 32768
GRID = (NUM_WORDS + BLK - 1) // BLK


def _sc_counts(x0):
    mesh = plsc.VectorSubcoreMesh(core_axis_name="c", subcore_axis_name="s")

    @functools.partial(
        pl.kernel,
        out_type=jax.ShapeDtypeStruct((2 * PAD_WORDS,), jnp.float32),
        mesh=mesh,
        scratch_types=[
            pltpu.VMEM((4, 128), jnp.int32),
            pltpu.VMEM((128,), jnp.float32),
            pltpu.VMEM((PER_TILE_WORDS,), jnp.float32),
            pltpu.VMEM_SHARED((NUM_WORDS,), jnp.float32),
        ],
    )
    def k(x0_hbm, cnt_hbm, idx_v, ones_v, z_v, cnt_s):
        cid = lax.axis_index("c")
        sid = lax.axis_index("s")
        wid = sid * 2 + cid
        base = wid * ROWS_PER_TILE

        for j in range(4):
            pltpu.sync_copy(x0_hbm.at[pl.ds(base + j * 128, 128)],
                            idx_v.at[j])

        one = jnp.full((16,), 1.0, jnp.float32)
        for j in range(8):
            ones_v[pl.ds(16 * j, 16)] = one

        zero = jnp.zeros((16,), jnp.float32)

        def zb(i, c):
            for j in range(16):
                z_v[pl.ds(i * 256 + j * 16, 16)] = zero
            return c

        lax.fori_loop(0, PER_TILE_WORDS // 256, zb, 0)

        tail = NUM_WORDS - 15 * PER_TILE_WORDS

        @pl.when(sid < 15)
        def _():
            pltpu.sync_copy(z_v, cnt_s.at[pl.ds(sid * PER_TILE_WORDS,
                                                PER_TILE_WORDS)])

        @pl.when(sid == 15)
        def _():
            pltpu.sync_copy(z_v.at[pl.ds(0, tail)],
                            cnt_s.at[pl.ds(15 * PER_TILE_WORDS, tail)])

        plsc.subcore_barrier()
        for j in range(4):
            pltpu.sync_copy(ones_v, cnt_s.at[idx_v.at[j]], add=True)
        plsc.subcore_barrier()

        cbase = cid * PAD_WORDS

        @pl.when(sid < 15)
        def _():
            pltpu.sync_copy(
                cnt_s.at[pl.ds(sid * PER_TILE_WORDS, PER_TILE_WORDS)], z_v)
            pltpu.sync_copy(
                z_v,
                cnt_hbm.at[pl.ds(cbase + sid * PER_TILE_WORDS,
                                 PER_TILE_WORDS)])

        @pl.when(sid == 15)
        def _():
            pltpu.sync_copy(
                z_v.at[pl.ds(0, PAD_WORDS - NUM_WORDS)],
                cnt_hbm.at[pl.ds(cbase + NUM_WORDS,
                                 PAD_WORDS - NUM_WORDS)])
            pltpu.sync_copy(
                cnt_s.at[pl.ds(15 * PER_TILE_WORDS, tail)],
                z_v.at[pl.ds(0, tail)])
            pltpu.sync_copy(
                z_v.at[pl.ds(0, tail)],
                cnt_hbm.at[pl.ds(cbase + 15 * PER_TILE_WORDS, tail)])

    return k(x0)


def _tc_stage(x1, counts, emb1_t, emb2_t, x2, W1, b1, W2, b2):
    def body(x1_ref, tbl_ref, cnt_ref, x2_ref, w1_ref, b1_ref, w2_ref,
             b2_ref, e2_hbm, o_ref, acc_ref, e2_v, sem):
        i = pl.program_id(0)

        @pl.when(i == 0)
        def _():
            acc_ref[...] = jnp.zeros_like(acc_ref)
            pltpu.async_copy(e2_hbm, e2_v, sem)

        acc_ref[...] += lax.dot_general(
            cnt_ref[...], tbl_ref[...], (((1,), (1,)), ((), ())),
            preferred_element_type=jnp.float32)

        @pl.when(i == GRID - 1)
        def _():
            s = jnp.sum(acc_ref[...], axis=0, keepdims=True) * (1.0 / DOC_LEN)
            y1 = lax.dot_general(s, w1_ref[...], (((1,), (1,)), ((), ())),
                                 preferred_element_type=jnp.float32)
            y1 = y1 + b1_ref[...]
            y3 = jax.nn.sigmoid(
                lax.dot_general(x2_ref[...], w2_ref[...],
                                (((1,), (1,)), ((), ())),
                                preferred_element_type=jnp.float32)
                + b2_ref[...])
            pltpu.make_async_copy(e2_hbm, e2_v, sem).wait()
            oh = (lax.broadcasted_iota(jnp.int32, (1, NUM_CP), 1)
                  == x1_ref[0]).astype(jnp.float32)
            y2 = lax.dot_general(oh, e2_v[...], (((1,), (1,)), ((), ())),
                                 preferred_element_type=jnp.float32)
            t = y2 + y3
            o_ref[...] = jax.nn.sigmoid(jnp.sum(y1 * t, axis=1,
                                                keepdims=True))

    grid_spec = pltpu.PrefetchScalarGridSpec(
        num_scalar_prefetch=1,
        grid=(GRID,),
        in_specs=[
            pl.BlockSpec((EMB, BLK), lambda i, x1r: (0, i)),
            pl.BlockSpec((2, BLK), lambda i, x1r: (0, i)),
            pl.BlockSpec((1, EMB), lambda i, x1r: (0, 0)),
            pl.BlockSpec((EMB, EMB), lambda i, x1r: (0, 0)),
            pl.BlockSpec((1, EMB), lambda i, x1r: (0, 0)),
            pl.BlockSpec((EMB, EMB), lambda i, x1r: (0, 0)),
            pl.BlockSpec((1, EMB), lambda i, x1r: (0, 0)),
            pl.BlockSpec(memory_space=pl.ANY),
        ],
        out_specs=pl.BlockSpec((1, 1), lambda i, x1r: (0, 0)),
        scratch_shapes=[
            pltpu.VMEM((2, EMB), jnp.float32),
            pltpu.VMEM((EMB, NUM_CP), jnp.float32),
            pltpu.SemaphoreType.DMA,
        ],
    )
    return pl.pallas_call(
        body,
        grid_spec=grid_spec,
        out_shape=jax.ShapeDtypeStruct((1, 1), jnp.float32),
        compiler_params=pltpu.CompilerParams(
            dimension_semantics=("arbitrary",)),
    )(x1, emb1_t, counts, x2, W1, b1, W2, b2, emb2_t)


def kernel(x0, x1, x2, emb1, emb2, W1, b1, W2, b2):
    counts = _sc_counts(x0).reshape(2, PAD_WORDS)
    out = _tc_stage(x1.astype(jnp.int32), counts, emb1.T, emb2.T,
                    x2.reshape(1, EMB), W1, b1.reshape(1, EMB),
                    W2, b2.reshape(1, EMB))
    return out.reshape(())


# speedup vs baseline: 3.3301x; 1.0460x over previous
import functools

import jax
import jax.numpy as jnp
from jax import lax
from jax.experimental import pallas as pl
from jax.experimental.pallas import tpu as pltpu
from jax.experimental.pallas import tpu_sc as plsc

DOC_LEN = 16384
EMB = 64
NUM_WORDS = 1000000
PAD_WORDS = 1 << 20
NUM_CP = 100000
NUM_TILES = 32
ROWS_PER_TILE = DOC_LEN // NUM_TILES
PER_TILE_WORDS = PAD_WORDS // 16
BLK = 65536
GRID = (NUM_WORDS + BLK - 1) // BLK


def _sc_counts(x0):
    mesh = plsc.VectorSubcoreMesh(core_axis_name="c", subcore_axis_name="s")

    @functools.partial(
        pl.kernel,
        out_type=jax.ShapeDtypeStruct((2 * PAD_WORDS,), jnp.float32),
        mesh=mesh,
        scratch_types=[
            pltpu.VMEM((4, 128), jnp.int32),
            pltpu.VMEM((128,), jnp.float32),
            pltpu.VMEM((PER_TILE_WORDS,), jnp.float32),
            pltpu.VMEM_SHARED((NUM_WORDS,), jnp.float32),
        ],
    )
    def k(x0_hbm, cnt_hbm, idx_v, ones_v, z_v, cnt_s):
        cid = lax.axis_index("c")
        sid = lax.axis_index("s")
        wid = sid * 2 + cid
        base = wid * ROWS_PER_TILE

        for j in range(4):
            pltpu.sync_copy(x0_hbm.at[pl.ds(base + j * 128, 128)],
                            idx_v.at[j])

        one = jnp.full((16,), 1.0, jnp.float32)
        for j in range(8):
            ones_v[pl.ds(16 * j, 16)] = one

        zero = jnp.zeros((16,), jnp.float32)

        def zb(i, c):
            for j in range(16):
                z_v[pl.ds(i * 256 + j * 16, 16)] = zero
            return c

        lax.fori_loop(0, PER_TILE_WORDS // 256, zb, 0)

        tail = NUM_WORDS - 15 * PER_TILE_WORDS

        @pl.when(sid < 15)
        def _():
            pltpu.sync_copy(z_v, cnt_s.at[pl.ds(sid * PER_TILE_WORDS,
                                                PER_TILE_WORDS)])

        @pl.when(sid == 15)
        def _():
            pltpu.sync_copy(z_v.at[pl.ds(0, tail)],
                            cnt_s.at[pl.ds(15 * PER_TILE_WORDS, tail)])

        plsc.subcore_barrier()
        for j in range(4):
            pltpu.sync_copy(ones_v, cnt_s.at[idx_v.at[j]], add=True)
        plsc.subcore_barrier()

        cbase = cid * PAD_WORDS

        @pl.when(sid < 15)
        def _():
            pltpu.sync_copy(
                cnt_s.at[pl.ds(sid * PER_TILE_WORDS, PER_TILE_WORDS)], z_v)
            pltpu.sync_copy(
                z_v,
                cnt_hbm.at[pl.ds(cbase + sid * PER_TILE_WORDS,
                                 PER_TILE_WORDS)])

        @pl.when(sid == 15)
        def _():
            pltpu.sync_copy(
                z_v.at[pl.ds(0, PAD_WORDS - NUM_WORDS)],
                cnt_hbm.at[pl.ds(cbase + NUM_WORDS,
                                 PAD_WORDS - NUM_WORDS)])
            pltpu.sync_copy(
                cnt_s.at[pl.ds(15 * PER_TILE_WORDS, tail)],
                z_v.at[pl.ds(0, tail)])
            pltpu.sync_copy(
                z_v.at[pl.ds(0, tail)],
                cnt_hbm.at[pl.ds(cbase + 15 * PER_TILE_WORDS, tail)])

    return k(x0)


def _tc_stage(x1, counts, emb1_t, emb2_t, x2, W1, b1, W2, b2):
    def body(x1_ref, tbl_ref, c0_ref, c1_ref, x2_ref, w1_ref, b1_ref,
             w2_ref, b2_ref, e2_hbm, o_ref, acc_ref, e2_v, sem):
        i = pl.program_id(0)

        @pl.when(i == 0)
        def _():
            acc_ref[...] = jnp.zeros_like(acc_ref)
            pltpu.async_copy(e2_hbm, e2_v, sem)

        c = (c0_ref[...] + c1_ref[...]).reshape(1, BLK)
        acc_ref[...] += lax.dot_general(
            c, tbl_ref[...], (((1,), (1,)), ((), ())),
            preferred_element_type=jnp.float32)

        @pl.when(i == GRID - 1)
        def _():
            s = acc_ref[...] * (1.0 / DOC_LEN)
            y1 = lax.dot_general(s, w1_ref[...], (((1,), (1,)), ((), ())),
                                 preferred_element_type=jnp.float32)
            y1 = y1 + b1_ref[...]
            y3 = jax.nn.sigmoid(
                lax.dot_general(x2_ref[...], w2_ref[...],
                                (((1,), (1,)), ((), ())),
                                preferred_element_type=jnp.float32)
                + b2_ref[...])
            pltpu.make_async_copy(e2_hbm, e2_v, sem).wait()
            oh = (lax.broadcasted_iota(jnp.int32, (1, NUM_CP), 1)
                  == x1_ref[0]).astype(jnp.float32)
            y2 = lax.dot_general(oh, e2_v[...], (((1,), (1,)), ((), ())),
                                 preferred_element_type=jnp.float32)
            t = y2 + y3
            o_ref[...] = jax.nn.sigmoid(jnp.sum(y1 * t, axis=1,
                                                keepdims=True))

    grid_spec = pltpu.PrefetchScalarGridSpec(
        num_scalar_prefetch=1,
        grid=(GRID,),
        in_specs=[
            pl.BlockSpec((EMB, BLK), lambda i, x1r: (0, i)),
            pl.BlockSpec((BLK,), lambda i, x1r: (i,)),
            pl.BlockSpec((BLK,), lambda i, x1r: (PAD_WORDS // BLK + i,)),
            pl.BlockSpec((1, EMB), lambda i, x1r: (0, 0)),
            pl.BlockSpec((EMB, EMB), lambda i, x1r: (0, 0)),
            pl.BlockSpec((1, EMB), lambda i, x1r: (0, 0)),
            pl.BlockSpec((EMB, EMB), lambda i, x1r: (0, 0)),
            pl.BlockSpec((1, EMB), lambda i, x1r: (0, 0)),
            pl.BlockSpec(memory_space=pl.ANY),
        ],
        out_specs=pl.BlockSpec((1, 1), lambda i, x1r: (0, 0)),
        scratch_shapes=[
            pltpu.VMEM((1, EMB), jnp.float32),
            pltpu.VMEM((EMB, NUM_CP), jnp.float32),
            pltpu.SemaphoreType.DMA,
        ],
    )
    return pl.pallas_call(
        body,
        grid_spec=grid_spec,
        out_shape=jax.ShapeDtypeStruct((1, 1), jnp.float32),
        compiler_params=pltpu.CompilerParams(
            dimension_semantics=("arbitrary",)),
    )(x1, emb1_t, counts, counts, x2, W1, b1, W2, b2, emb2_t)


def kernel(x0, x1, x2, emb1, emb2, W1, b1, W2, b2):
    counts = _sc_counts(x0)
    out = _tc_stage(x1.astype(jnp.int32), counts, emb1.T, emb2.T,
                    x2.reshape(1, EMB), W1, b1.reshape(1, EMB),
                    W2, b2.reshape(1, EMB))
    return out.reshape(())


# speedup vs baseline: 3.4935x; 1.0491x over previous
import functools

import jax
import jax.numpy as jnp
from jax import lax
from jax.experimental import pallas as pl
from jax.experimental.pallas import tpu as pltpu
from jax.experimental.pallas import tpu_sc as plsc

DOC_LEN = 16384
EMB = 64
NUM_WORDS = 1000000
PAD_WORDS = 1 << 20
NUM_CP = 100000
NUM_TILES = 32
ROWS_PER_TILE = DOC_LEN // NUM_TILES
PER_TILE_WORDS = PAD_WORDS // 16
BLK = 32768
GRID = (NUM_WORDS + BLK - 1) // BLK


def _sc_counts(x0):
    mesh = plsc.VectorSubcoreMesh(core_axis_name="c", subcore_axis_name="s")

    @functools.partial(
        pl.kernel,
        out_type=jax.ShapeDtypeStruct((2 * PAD_WORDS,), jnp.float32),
        mesh=mesh,
        scratch_types=[
            pltpu.VMEM((4, 128), jnp.int32),
            pltpu.VMEM((128,), jnp.float32),
            pltpu.VMEM((PER_TILE_WORDS,), jnp.float32),
            pltpu.VMEM_SHARED((NUM_WORDS,), jnp.float32),
        ],
    )
    def k(x0_hbm, cnt_hbm, idx_v, ones_v, z_v, cnt_s):
        cid = lax.axis_index("c")
        sid = lax.axis_index("s")
        wid = sid * 2 + cid
        base = wid * ROWS_PER_TILE

        for j in range(4):
            pltpu.sync_copy(x0_hbm.at[pl.ds(base + j * 128, 128)],
                            idx_v.at[j])

        one = jnp.full((16,), 1.0, jnp.float32)
        for j in range(8):
            ones_v[pl.ds(16 * j, 16)] = one

        zero = jnp.zeros((16,), jnp.float32)

        def zb(i, c):
            for j in range(16):
                z_v[pl.ds(i * 256 + j * 16, 16)] = zero
            return c

        lax.fori_loop(0, PER_TILE_WORDS // 256, zb, 0)

        tail = NUM_WORDS - 15 * PER_TILE_WORDS

        @pl.when(sid < 15)
        def _():
            pltpu.sync_copy(z_v, cnt_s.at[pl.ds(sid * PER_TILE_WORDS,
                                                PER_TILE_WORDS)])

        @pl.when(sid == 15)
        def _():
            pltpu.sync_copy(z_v.at[pl.ds(0, tail)],
                            cnt_s.at[pl.ds(15 * PER_TILE_WORDS, tail)])

        plsc.subcore_barrier()
        for j in range(4):
            pltpu.sync_copy(ones_v, cnt_s.at[idx_v.at[j]], add=True)
        plsc.subcore_barrier()

        cbase = cid * PAD_WORDS

        @pl.when(sid < 15)
        def _():
            pltpu.sync_copy(
                cnt_s.at[pl.ds(sid * PER_TILE_WORDS, PER_TILE_WORDS)], z_v)
            pltpu.sync_copy(
                z_v,
                cnt_hbm.at[pl.ds(cbase + sid * PER_TILE_WORDS,
                                 PER_TILE_WORDS)])

        @pl.when(sid == 15)
        def _():
            pltpu.sync_copy(
                z_v.at[pl.ds(0, PAD_WORDS - NUM_WORDS)],
                cnt_hbm.at[pl.ds(cbase + NUM_WORDS,
                                 PAD_WORDS - NUM_WORDS)])
            pltpu.sync_copy(
                cnt_s.at[pl.ds(15 * PER_TILE_WORDS, tail)],
                z_v.at[pl.ds(0, tail)])
            pltpu.sync_copy(
                z_v.at[pl.ds(0, tail)],
                cnt_hbm.at[pl.ds(cbase + 15 * PER_TILE_WORDS, tail)])

    return k(x0)


def _tc_stage(x1, counts, emb1_t, emb2_t, x2, W1, b1, W2, b2):
    def body(x1_ref, tbl_ref, c0_ref, c1_ref, x2_ref, w1_ref, b1_ref,
             w2_ref, b2_ref, e2_hbm, o_ref, acc_ref, e2_v, sem):
        i = pl.program_id(0)

        @pl.when(i == 0)
        def _():
            acc_ref[...] = jnp.zeros_like(acc_ref)
            pltpu.async_copy(e2_hbm, e2_v, sem)

        c = (c0_ref[...] + c1_ref[...]).reshape(1, BLK)
        acc_ref[...] += lax.dot_general(
            c, tbl_ref[...], (((1,), (1,)), ((), ())),
            preferred_element_type=jnp.float32)

        @pl.when(i == GRID - 1)
        def _():
            s = acc_ref[...] * (1.0 / DOC_LEN)
            y1 = lax.dot_general(s, w1_ref[...], (((1,), (1,)), ((), ())),
                                 preferred_element_type=jnp.float32)
            y1 = y1 + b1_ref[...]
            y3 = jax.nn.sigmoid(
                lax.dot_general(x2_ref[...], w2_ref[...],
                                (((1,), (1,)), ((), ())),
                                preferred_element_type=jnp.float32)
                + b2_ref[...])
            pltpu.make_async_copy(e2_hbm, e2_v, sem).wait()
            oh = (lax.broadcasted_iota(jnp.int32, (1, NUM_CP), 1)
                  == x1_ref[0]).astype(jnp.float32)
            y2 = lax.dot_general(oh, e2_v[...], (((1,), (1,)), ((), ())),
                                 preferred_element_type=jnp.float32)
            t = y2 + y3
            o_ref[...] = jax.nn.sigmoid(jnp.sum(y1 * t, axis=1,
                                                keepdims=True))

    grid_spec = pltpu.PrefetchScalarGridSpec(
        num_scalar_prefetch=1,
        grid=(GRID,),
        in_specs=[
            pl.BlockSpec((EMB, BLK), lambda i, x1r: (0, i)),
            pl.BlockSpec((BLK,), lambda i, x1r: (i,)),
            pl.BlockSpec((BLK,), lambda i, x1r: (PAD_WORDS // BLK + i,)),
            pl.BlockSpec((1, EMB), lambda i, x1r: (0, 0)),
            pl.BlockSpec((EMB, EMB), lambda i, x1r: (0, 0)),
            pl.BlockSpec((1, EMB), lambda i, x1r: (0, 0)),
            pl.BlockSpec((EMB, EMB), lambda i, x1r: (0, 0)),
            pl.BlockSpec((1, EMB), lambda i, x1r: (0, 0)),
            pl.BlockSpec(memory_space=pl.ANY),
        ],
        out_specs=pl.BlockSpec((1, 1), lambda i, x1r: (0, 0)),
        scratch_shapes=[
            pltpu.VMEM((1, EMB), jnp.float32),
            pltpu.VMEM((EMB, NUM_CP), jnp.float32),
            pltpu.SemaphoreType.DMA,
        ],
    )
    return pl.pallas_call(
        body,
        grid_spec=grid_spec,
        out_shape=jax.ShapeDtypeStruct((1, 1), jnp.float32),
        compiler_params=pltpu.CompilerParams(
            dimension_semantics=("arbitrary",)),
    )(x1, emb1_t, counts, counts, x2, W1, b1, W2, b2, emb2_t)


def kernel(x0, x1, x2, emb1, emb2, W1, b1, W2, b2):
    counts = _sc_counts(x0)
    out = _tc_stage(x1.astype(jnp.int32), counts, emb1.T, emb2.T,
                    x2.reshape(1, EMB), W1, b1.reshape(1, EMB),
                    W2, b2.reshape(1, EMB))
    return out.reshape(())
